# baseline (speedup 1.0000x reference)
import jax
import jax.numpy as jnp
import numpy as np
from jax import lax
from jax.experimental import pallas as pl
from jax.experimental.pallas import tpu as pltpu
from jax.experimental.pallas import tpu_sc as plsc

NUM_STATE = 128
V = 100000
B = 16384

RB = 20000
NRB = V // RB

NC, NS = 2, 16
NW = NC * NS
BPW = B // NW
CHUNK = 128
NCHUNK = BPW // CHUNK

_PEN_TABLE = np.concatenate(
    [np.zeros((CHUNK, NUM_STATE), np.float32),
     np.full((CHUNK, NUM_STATE), -1e30, np.float32)], axis=0
)

FB = 4096


def _sc_gather(ut_hbm, idx_hbm, pen_hbm, raw_hbm,
               idx_v, idx_c, flag_v, rows_v, pen_sp, sem, sem2, sem3):
    wid = lax.axis_index("s") * NC + lax.axis_index("c")
    base = wid * BPW

    @pl.when(lax.axis_index("s") == 0)
    def _stage_pen():
        pltpu.sync_copy(pen_hbm, pen_sp)

    pltpu.sync_copy(idx_hbm.at[pl.ds(base, BPW)], idx_v)
    lanes = lax.iota(jnp.int32, 16)

    def _prep(g, carry):
        sl = pl.ds(g * 16, 16)
        raw_idx = idx_v[sl]
        idx_c[sl] = jnp.minimum(raw_idx, V - 1)
        offs = ((g * 16) % CHUNK) + lanes
        flag_v[sl] = jnp.where(raw_idx >= V, CHUNK, 0) + offs
        return carry

    lax.fori_loop(0, BPW // 16, _prep, 0)
    copies = [
        pltpu.async_copy(
            ut_hbm.at[idx_c.at[pl.ds(k * CHUNK, CHUNK)]],
            rows_v.at[pl.ds(k * CHUNK, CHUNK)],
            sem,
        )
        for k in range(NCHUNK)
    ]
    plsc.subcore_barrier()
    adds = []
    for k in range(NCHUNK):
        copies[k].wait()
        adds.append(
            pltpu.async_copy(
                pen_sp.at[flag_v.at[pl.ds(k * CHUNK, CHUNK)]],
                rows_v.at[pl.ds(k * CHUNK, CHUNK)],
                sem2,
                add=True,
            )
        )
    outs = []
    for k in range(NCHUNK):
        adds[k].wait()
        outs.append(
            pltpu.async_copy(
                rows_v.at[pl.ds(k * CHUNK, CHUNK)],
                raw_hbm.at[pl.ds(base + k * CHUNK, CHUNK)],
                sem3,
            )
        )
    for o in outs:
        o.wait()


def _tc_sumexp(ut_ref, rinv_ref, acc_ref):
    i = pl.program_id(0)

    @pl.when(i == 0)
    def _init():
        acc_ref[...] = jnp.zeros_like(acc_ref)

    acc_ref[...] += jnp.sum(jnp.exp(ut_ref[...]), axis=0, keepdims=True)

    @pl.when(i == NRB - 1)
    def _fin():
        rinv_ref[...] = 1.0 / acc_ref[...]


def _tc_finalize(raw_ref, rinv_ref, out_ref):
    out_ref[...] = jnp.exp(raw_ref[...]) * rinv_ref[...]


def kernel(state_embeddings, observation_embeddings, x_t, unnormalized_emission_matrix):
    del state_embeddings, observation_embeddings
    ut = unnormalized_emission_matrix.T
    x_i32 = x_t.astype(jnp.int32)
    pen = jnp.asarray(_PEN_TABLE)

    mesh = plsc.VectorSubcoreMesh(
        core_axis_name="c", subcore_axis_name="s", num_cores=NC, num_subcores=NS
    )
    raw = pl.kernel(
        _sc_gather,
        out_type=jax.ShapeDtypeStruct((B, NUM_STATE), jnp.float32),
        mesh=mesh,
        scratch_types=[
            pltpu.VMEM((BPW,), jnp.int32),
            pltpu.VMEM((BPW,), jnp.int32),
            pltpu.VMEM((BPW,), jnp.int32),
            pltpu.VMEM((BPW, NUM_STATE), jnp.float32),
            pltpu.VMEM_SHARED((2 * CHUNK, NUM_STATE), jnp.float32),
            pltpu.SemaphoreType.DMA,
            pltpu.SemaphoreType.DMA,
            pltpu.SemaphoreType.DMA,
        ],
    )(ut, x_i32, pen)

    rinv = pl.pallas_call(
        _tc_sumexp,
        grid=(NRB,),
        in_specs=[pl.BlockSpec((RB, NUM_STATE), lambda i: (i, 0))],
        out_specs=pl.BlockSpec((1, NUM_STATE), lambda i: (0, 0)),
        out_shape=jax.ShapeDtypeStruct((1, NUM_STATE), jnp.float32),
        scratch_shapes=[pltpu.VMEM((1, NUM_STATE), jnp.float32)],
    )(ut)

    return pl.pallas_call(
        _tc_finalize,
        grid=(B // FB,),
        in_specs=[
            pl.BlockSpec((FB, NUM_STATE), lambda i: (i, 0)),
            pl.BlockSpec((1, NUM_STATE), lambda i: (0, 0)),
        ],
        out_specs=pl.BlockSpec((FB, NUM_STATE), lambda i: (i, 0)),
        out_shape=jax.ShapeDtypeStruct((B, NUM_STATE), jnp.float32),
    )(raw, rinv)


# speedup vs baseline: 1.0359x; 1.0359x over previous
import jax
import jax.numpy as jnp
import numpy as np
from jax import lax
from jax.experimental import pallas as pl
from jax.experimental.pallas import tpu as pltpu
from jax.experimental.pallas import tpu_sc as plsc

NUM_STATE = 128
V = 100000
B = 16384

RB = 20000
NRB = V // RB

NC, NS = 2, 16
NW = NC * NS
BPW = B // NW
CHUNK = 128
NCHUNK = BPW // CHUNK

_PEN_TABLE = np.concatenate(
    [np.zeros((CHUNK, NUM_STATE), np.float32),
     np.full((CHUNK, NUM_STATE), -1e30, np.float32)], axis=0
)

FB = 8192


def _sc_gather(ut_hbm, idx_hbm, pen_hbm, raw_hbm,
               idx_v, idx_c, flag_v, rows_v, pen_sp, sem, sem2, sem3):
    wid = lax.axis_index("s") * NC + lax.axis_index("c")
    base = wid * BPW

    @pl.when(lax.axis_index("s") == 0)
    def _stage_pen():
        pltpu.sync_copy(pen_hbm, pen_sp)

    pltpu.sync_copy(idx_hbm.at[pl.ds(base, BPW)], idx_v)
    lanes = lax.iota(jnp.int32, 16)

    def _prep(g, carry):
        sl = pl.ds(g * 16, 16)
        raw_idx = idx_v[sl]
        idx_c[sl] = jnp.minimum(raw_idx, V - 1)
        offs = ((g * 16) % CHUNK) + lanes
        flag_v[sl] = jnp.where(raw_idx >= V, CHUNK, 0) + offs
        return carry

    lax.fori_loop(0, BPW // 16, _prep, 0)
    copies = [
        pltpu.async_copy(
            ut_hbm.at[idx_c.at[pl.ds(k * CHUNK, CHUNK)]],
            rows_v.at[pl.ds(k * CHUNK, CHUNK)],
            sem,
        )
        for k in range(NCHUNK)
    ]
    plsc.subcore_barrier()
    adds = []
    for k in range(NCHUNK):
        copies[k].wait()
        adds.append(
            pltpu.async_copy(
                pen_sp.at[flag_v.at[pl.ds(k * CHUNK, CHUNK)]],
                rows_v.at[pl.ds(k * CHUNK, CHUNK)],
                sem2,
                add=True,
            )
        )
    outs = []
    for k in range(NCHUNK):
        adds[k].wait()
        outs.append(
            pltpu.async_copy(
                rows_v.at[pl.ds(k * CHUNK, CHUNK)],
                raw_hbm.at[pl.ds(base + k * CHUNK, CHUNK)],
                sem3,
            )
        )
    for o in outs:
        o.wait()


def _tc_sumexp(ut_ref, rinv_ref, acc_ref):
    i = pl.program_id(0)

    @pl.when(i == 0)
    def _init():
        acc_ref[...] = jnp.zeros_like(acc_ref)

    acc_ref[...] += jnp.sum(jnp.exp(ut_ref[...]), axis=0, keepdims=True)

    @pl.when(i == NRB - 1)
    def _fin():
        rinv_ref[...] = 1.0 / acc_ref[...]


def _tc_finalize(raw_ref, rinv_ref, out_ref):
    out_ref[...] = jnp.exp(raw_ref[...]) * rinv_ref[...]


def kernel(state_embeddings, observation_embeddings, x_t, unnormalized_emission_matrix):
    del state_embeddings, observation_embeddings
    ut = unnormalized_emission_matrix.T
    x_i32 = x_t.astype(jnp.int32)
    pen = jnp.asarray(_PEN_TABLE)

    mesh = plsc.VectorSubcoreMesh(
        core_axis_name="c", subcore_axis_name="s", num_cores=NC, num_subcores=NS
    )
    raw = pl.kernel(
        _sc_gather,
        out_type=jax.ShapeDtypeStruct((B, NUM_STATE), jnp.float32),
        mesh=mesh,
        scratch_types=[
            pltpu.VMEM((BPW,), jnp.int32),
            pltpu.VMEM((BPW,), jnp.int32),
            pltpu.VMEM((BPW,), jnp.int32),
            pltpu.VMEM((BPW, NUM_STATE), jnp.float32),
            pltpu.VMEM_SHARED((2 * CHUNK, NUM_STATE), jnp.float32),
            pltpu.SemaphoreType.DMA,
            pltpu.SemaphoreType.DMA,
            pltpu.SemaphoreType.DMA,
        ],
    )(ut, x_i32, pen)

    rinv = pl.pallas_call(
        _tc_sumexp,
        grid=(NRB,),
        in_specs=[pl.BlockSpec((RB, NUM_STATE), lambda i: (i, 0))],
        out_specs=pl.BlockSpec((1, NUM_STATE), lambda i: (0, 0)),
        out_shape=jax.ShapeDtypeStruct((1, NUM_STATE), jnp.float32),
        scratch_shapes=[pltpu.VMEM((1, NUM_STATE), jnp.float32)],
    )(ut)

    return pl.pallas_call(
        _tc_finalize,
        grid=(B // FB,),
        in_specs=[
            pl.BlockSpec((FB, NUM_STATE), lambda i: (i, 0)),
            pl.BlockSpec((1, NUM_STATE), lambda i: (0, 0)),
        ],
        out_specs=pl.BlockSpec((FB, NUM_STATE), lambda i: (i, 0)),
        out_shape=jax.ShapeDtypeStruct((B, NUM_STATE), jnp.float32),
    )(raw, rinv)
